# baseline (speedup 1.0000x reference)
import jax
import jax.numpy as jnp
from jax.experimental import pallas as pl
from jax.experimental.pallas import tpu as pltpu

B, N, M, DIM = 16, 2048, 2048, 3
TI = 256
NI = N // TI


def _chamfer_body(p1_ref, p2t_ref, out_ref, cmin_ref):
    b = pl.program_id(0)
    it = pl.program_id(1)
    p1 = p1_ref[0]
    p2t = p2t_ref[0]

    sq1 = jnp.sum(p1 * p1, axis=1, keepdims=True)
    sq2 = jnp.sum(p2t * p2t, axis=0, keepdims=True)
    inner2 = jax.lax.dot_general(
        p1 * (-2.0), p2t, (((1,), (0,)), ((), ())),
        preferred_element_type=jnp.float32)
    d_tile = (sq1 + sq2) + inner2

    rowmin = jnp.maximum(jnp.min(d_tile, axis=1), 0.0)
    colmin = jnp.min(d_tile, axis=0, keepdims=True)

    @pl.when(jnp.logical_and(b == 0, it == 0))
    def _init():
        out_ref[...] = jnp.zeros_like(out_ref)

    @pl.when(it == 0)
    def _reset():
        cmin_ref[...] = colmin

    @pl.when(it != 0)
    def _accmin():
        cmin_ref[...] = jnp.minimum(cmin_ref[...], colmin)

    acc = jnp.sum(rowmin) * (1.0 / (B * N))

    @pl.when(it != NI - 1)
    def _accrow():
        out_ref[...] += acc

    @pl.when(it == NI - 1)
    def _final():
        colsum = jnp.sum(jnp.maximum(cmin_ref[...], 0.0))
        out_ref[...] += acc + colsum * (1.0 / (B * M))


def kernel(pcs1, pcs2):
    p2t = jnp.transpose(pcs2, (0, 2, 1))
    out = pl.pallas_call(
        _chamfer_body,
        grid=(B, NI),
        in_specs=[
            pl.BlockSpec((1, TI, DIM), lambda b, i: (b, i, 0)),
            pl.BlockSpec((1, DIM, M), lambda b, i: (b, 0, 0)),
        ],
        out_specs=pl.BlockSpec((1, 1), lambda b, i: (0, 0)),
        out_shape=jax.ShapeDtypeStruct((1, 1), jnp.float32),
        scratch_shapes=[pltpu.VMEM((1, M), jnp.float32)],
        compiler_params=pltpu.CompilerParams(
            dimension_semantics=("arbitrary", "arbitrary")),
    )(pcs1, p2t)
    return out[0, 0]


# speedup vs baseline: 1.0113x; 1.0113x over previous
import jax
import jax.numpy as jnp
from jax.experimental import pallas as pl
from jax.experimental.pallas import tpu as pltpu

B, N, M, DIM = 16, 2048, 2048, 3
TI = 256
NI = N // TI


def _chamfer_body(p1_ref, p2t_ref, out_ref, cmin_ref, rowacc_ref, colacc_ref):
    b = pl.program_id(0)
    it = pl.program_id(1)
    first = jnp.logical_and(b == 0, it == 0)
    p1 = p1_ref[0]
    p2t = p2t_ref[0]

    sq1 = jnp.sum(p1 * p1, axis=1, keepdims=True)
    sq2 = jnp.sum(p2t * p2t, axis=0, keepdims=True)
    one1 = jnp.ones_like(sq1)
    one2 = jnp.ones_like(sq2)
    sq1_hi = sq1.astype(jnp.bfloat16).astype(jnp.float32)
    sq2_hi = sq2.astype(jnp.bfloat16).astype(jnp.float32)
    lhs = jnp.concatenate(
        [p1 * (-2.0), sq1_hi, sq1 - sq1_hi, one1, one1], axis=1)
    rhs = jnp.concatenate(
        [p2t, one2, one2, sq2_hi, sq2 - sq2_hi], axis=0)
    d = jax.lax.dot_general(
        lhs.astype(jnp.bfloat16), rhs.astype(jnp.bfloat16),
        (((1,), (0,)), ((), ())),
        preferred_element_type=jnp.float32)

    rm = jnp.min(d, axis=1)[None, :]
    prev_row = jnp.where(first, jnp.zeros_like(rowacc_ref[...]),
                         rowacc_ref[...])
    rowacc_ref[...] = prev_row + jnp.maximum(rm, 0.0)

    cm = jnp.min(d, axis=0, keepdims=True)
    prev_cm = jnp.where(it == 0, jnp.full_like(cm, jnp.inf), cmin_ref[...])
    cminv = jnp.minimum(prev_cm, cm)
    cmin_ref[...] = cminv

    last_it = it == NI - 1
    prev_col = jnp.where(first, jnp.zeros_like(colacc_ref[...]),
                         colacc_ref[...])
    colacc_ref[...] = prev_col + jnp.where(
        last_it, jnp.maximum(cminv, 0.0), jnp.zeros_like(cminv))

    @pl.when(jnp.logical_and(b == B - 1, last_it))
    def _final():
        total = (jnp.sum(rowacc_ref[...]) * (1.0 / (B * N))
                 + jnp.sum(colacc_ref[...]) * (1.0 / (B * M)))
        out_ref[...] = jnp.full_like(out_ref, total)


def kernel(pcs1, pcs2):
    p2t = jnp.transpose(pcs2, (0, 2, 1))
    out = pl.pallas_call(
        _chamfer_body,
        grid=(B, NI),
        in_specs=[
            pl.BlockSpec((1, TI, DIM), lambda b, i: (b, i, 0)),
            pl.BlockSpec((1, DIM, M), lambda b, i: (b, 0, 0)),
        ],
        out_specs=pl.BlockSpec((1, 1), lambda b, i: (0, 0)),
        out_shape=jax.ShapeDtypeStruct((1, 1), jnp.float32),
        scratch_shapes=[
            pltpu.VMEM((1, M), jnp.float32),
            pltpu.VMEM((1, TI), jnp.float32),
            pltpu.VMEM((1, M), jnp.float32),
        ],
        compiler_params=pltpu.CompilerParams(
            dimension_semantics=("arbitrary", "arbitrary")),
    )(pcs1, p2t)
    return out[0, 0]


# speedup vs baseline: 2.0585x; 2.0354x over previous
import jax
import jax.numpy as jnp
from jax.experimental import pallas as pl
from jax.experimental.pallas import tpu as pltpu

B, N, M, DIM = 16, 2048, 2048, 3
TI = 256
NI = N // TI


def _chamfer_body(p1_ref, p2t_ref, out_ref, rowacc_ref, colacc_ref):
    b = pl.program_id(0)
    p2t = p2t_ref[0]

    sq2 = jnp.sum(p2t * p2t, axis=0, keepdims=True)
    one2 = jnp.ones_like(sq2)
    sq2_hi = sq2.astype(jnp.bfloat16).astype(jnp.float32)
    rhs = jnp.concatenate(
        [p2t, one2, one2, sq2_hi, sq2 - sq2_hi],
        axis=0).astype(jnp.bfloat16)

    rs = None
    cm = None
    for it in range(NI):
        p1 = p1_ref[0, it * TI:(it + 1) * TI, :]
        sq1 = jnp.sum(p1 * p1, axis=1, keepdims=True)
        one1 = jnp.ones_like(sq1)
        sq1_hi = sq1.astype(jnp.bfloat16).astype(jnp.float32)
        lhs = jnp.concatenate(
            [p1 * (-2.0), sq1_hi, sq1 - sq1_hi, one1, one1],
            axis=1).astype(jnp.bfloat16)
        d = jax.lax.dot_general(
            lhs, rhs, (((1,), (0,)), ((), ())),
            preferred_element_type=jnp.float32)
        rm = jnp.maximum(jnp.min(d, axis=1), 0.0)[None, :]
        rs = rm if rs is None else rs + rm
        cmt = jnp.min(d, axis=0, keepdims=True)
        cm = cmt if cm is None else jnp.minimum(cm, cmt)

    first = b == 0
    prev_row = jnp.where(first, jnp.zeros_like(rowacc_ref[...]),
                         rowacc_ref[...])
    rowacc_ref[...] = prev_row + rs
    prev_col = jnp.where(first, jnp.zeros_like(colacc_ref[...]),
                         colacc_ref[...])
    colacc_ref[...] = prev_col + jnp.maximum(cm, 0.0)

    @pl.when(b == B - 1)
    def _final():
        total = (jnp.sum(rowacc_ref[...]) * (1.0 / (B * N))
                 + jnp.sum(colacc_ref[...]) * (1.0 / (B * M)))
        out_ref[...] = jnp.full_like(out_ref, total)


def kernel(pcs1, pcs2):
    p2t = jnp.transpose(pcs2, (0, 2, 1))
    out = pl.pallas_call(
        _chamfer_body,
        grid=(B,),
        in_specs=[
            pl.BlockSpec((1, N, DIM), lambda b: (b, 0, 0)),
            pl.BlockSpec((1, DIM, M), lambda b: (b, 0, 0)),
        ],
        out_specs=pl.BlockSpec((1, 1), lambda b: (0, 0)),
        out_shape=jax.ShapeDtypeStruct((1, 1), jnp.float32),
        scratch_shapes=[
            pltpu.VMEM((1, TI), jnp.float32),
            pltpu.VMEM((1, M), jnp.float32),
        ],
        compiler_params=pltpu.CompilerParams(
            dimension_semantics=("arbitrary",)),
    )(pcs1, p2t)
    return out[0, 0]


# speedup vs baseline: 2.1810x; 1.0595x over previous
import jax
import jax.numpy as jnp
from jax.experimental import pallas as pl
from jax.experimental.pallas import tpu as pltpu

B, N, M, DIM = 16, 2048, 2048, 3
TI = 256
NI = N // TI


BB = 4
GS = B // BB


def _chamfer_body(p1_ref, p2t_ref, out_ref, rowacc_ref, colacc_ref):
    g = pl.program_id(0)

    rs = None
    cs = None
    for bb in range(BB):
        p2t = p2t_ref[bb]
        sq2 = jnp.sum(p2t * p2t, axis=0, keepdims=True)
        one2 = jnp.ones_like(sq2)
        sq2_hi = sq2.astype(jnp.bfloat16).astype(jnp.float32)
        rhs = jnp.concatenate(
            [p2t, one2, one2, sq2_hi, sq2 - sq2_hi],
            axis=0).astype(jnp.bfloat16)

        cm = None
        for it in range(NI):
            p1 = p1_ref[bb, it * TI:(it + 1) * TI, :]
            sq1 = jnp.sum(p1 * p1, axis=1, keepdims=True)
            one1 = jnp.ones_like(sq1)
            sq1_hi = sq1.astype(jnp.bfloat16).astype(jnp.float32)
            lhs = jnp.concatenate(
                [p1 * (-2.0), sq1_hi, sq1 - sq1_hi, one1, one1],
                axis=1).astype(jnp.bfloat16)
            d = jax.lax.dot_general(
                lhs, rhs, (((1,), (0,)), ((), ())),
                preferred_element_type=jnp.float32)
            rm = jnp.maximum(jnp.min(d, axis=1), 0.0)[None, :]
            rs = rm if rs is None else rs + rm
            cmt = jnp.min(d, axis=0, keepdims=True)
            cm = cmt if cm is None else jnp.minimum(cm, cmt)
        cmc = jnp.maximum(cm, 0.0)
        cs = cmc if cs is None else cs + cmc

    first = g == 0
    prev_row = jnp.where(first, jnp.zeros_like(rowacc_ref[...]),
                         rowacc_ref[...])
    rowacc_ref[...] = prev_row + rs
    prev_col = jnp.where(first, jnp.zeros_like(colacc_ref[...]),
                         colacc_ref[...])
    colacc_ref[...] = prev_col + cs

    @pl.when(g == GS - 1)
    def _final():
        total = (jnp.sum(rowacc_ref[...]) * (1.0 / (B * N))
                 + jnp.sum(colacc_ref[...]) * (1.0 / (B * M)))
        out_ref[...] = jnp.full_like(out_ref, total)


def kernel(pcs1, pcs2):
    p2t = jnp.transpose(pcs2, (0, 2, 1))
    out = pl.pallas_call(
        _chamfer_body,
        grid=(GS,),
        in_specs=[
            pl.BlockSpec((BB, N, DIM), lambda g: (g, 0, 0)),
            pl.BlockSpec((BB, DIM, M), lambda g: (g, 0, 0)),
        ],
        out_specs=pl.BlockSpec((1, 1), lambda g: (0, 0)),
        out_shape=jax.ShapeDtypeStruct((1, 1), jnp.float32),
        scratch_shapes=[
            pltpu.VMEM((1, TI), jnp.float32),
            pltpu.VMEM((1, M), jnp.float32),
        ],
        compiler_params=pltpu.CompilerParams(
            dimension_semantics=("arbitrary",)),
    )(pcs1, p2t)
    return out[0, 0]
